# Initial kernel scaffold; baseline (speedup 1.0000x reference)
import jax
import jax.numpy as jnp
from jax.experimental import pallas as pl


def kernel(x_atm, x_bnd, x_ang, edge_index_G, edge_index_A, x_atm_batch, embed_atm, Wa, ba, Wb, bb, W_head, b_head, W_out, b_out):
    raise NotImplementedError("write your pallas kernel here")


# speedup vs baseline: 1.2480x; 1.2480x over previous
import functools

import jax
import jax.numpy as jnp
from jax.experimental import pallas as pl

DIM = 128
CUT = 3.0


def _silu(x):
    return x * jax.nn.sigmoid(x)


def _node_transform_body(h_ref, w_ref, b_ref, p0_ref, p1_ref, p4_ref):
    h = h_ref[...]
    w = w_ref[...]
    b = b_ref[...]
    p0_ref[...] = jnp.dot(h, w[0], preferred_element_type=jnp.float32) + b[0][None, :]
    p1_ref[...] = jnp.dot(h, w[1], preferred_element_type=jnp.float32) + b[1][None, :]
    p4_ref[...] = jnp.dot(h, w[2], preferred_element_type=jnp.float32) + b[2][None, :]


def _node_transform(h, w3, b3, block):
    n = h.shape[0]
    grid = (n // block,)
    out = pl.pallas_call(
        _node_transform_body,
        grid=grid,
        in_specs=[
            pl.BlockSpec((block, DIM), lambda i: (i, 0)),
            pl.BlockSpec((3, DIM, DIM), lambda i: (0, 0, 0)),
            pl.BlockSpec((3, DIM), lambda i: (0, 0)),
        ],
        out_specs=[
            pl.BlockSpec((block, DIM), lambda i: (i, 0)),
            pl.BlockSpec((block, DIM), lambda i: (i, 0)),
            pl.BlockSpec((block, DIM), lambda i: (i, 0)),
        ],
        out_shape=[jax.ShapeDtypeStruct((n, DIM), jnp.float32)] * 3,
    )(h, w3, b3)
    return out


def _edge_update_body(g0_ref, g1_ref, g4_ref, e_ref, w2_ref, b2_ref,
                      m_ref, enew_ref):
    e = e_ref[...]
    z = (g0_ref[...] + g1_ref[...]
         + jnp.dot(e, w2_ref[...], preferred_element_type=jnp.float32)
         + b2_ref[...][None, :])
    sigma = jax.nn.sigmoid(z)
    m_ref[...] = jnp.concatenate([sigma * g4_ref[...], sigma], axis=1)
    enew_ref[...] = e + _silu(z)


def _edge_update(g0, g1, g4, e, w2, b2, block):
    n = e.shape[0]
    grid = (n // block,)
    return pl.pallas_call(
        _edge_update_body,
        grid=grid,
        in_specs=[
            pl.BlockSpec((block, DIM), lambda i: (i, 0)),
            pl.BlockSpec((block, DIM), lambda i: (i, 0)),
            pl.BlockSpec((block, DIM), lambda i: (i, 0)),
            pl.BlockSpec((block, DIM), lambda i: (i, 0)),
            pl.BlockSpec((DIM, DIM), lambda i: (0, 0)),
            pl.BlockSpec((DIM,), lambda i: (0,)),
        ],
        out_specs=[
            pl.BlockSpec((block, 2 * DIM), lambda i: (i, 0)),
            pl.BlockSpec((block, DIM), lambda i: (i, 0)),
        ],
        out_shape=[
            jax.ShapeDtypeStruct((n, 2 * DIM), jnp.float32),
            jax.ShapeDtypeStruct((n, DIM), jnp.float32),
        ],
    )(g0, g1, g4, e, w2, b2)


def _node_update_body(h_ref, an_ref, w3_ref, b3_ref, out_ref):
    h = h_ref[...]
    an = an_ref[...]
    agg = an[:, :DIM]
    norm = an[:, DIM:]
    s3 = jnp.dot(h, w3_ref[...], preferred_element_type=jnp.float32) + b3_ref[...][None, :]
    out_ref[...] = h + _silu(s3 + agg / (norm + 1e-6))


def _node_update(h, aggnorm, w3, b3, block):
    n = h.shape[0]
    grid = (n // block,)
    return pl.pallas_call(
        _node_update_body,
        grid=grid,
        in_specs=[
            pl.BlockSpec((block, DIM), lambda i: (i, 0)),
            pl.BlockSpec((block, 2 * DIM), lambda i: (i, 0)),
            pl.BlockSpec((DIM, DIM), lambda i: (0, 0)),
            pl.BlockSpec((DIM,), lambda i: (0,)),
        ],
        out_specs=pl.BlockSpec((block, DIM), lambda i: (i, 0)),
        out_shape=jax.ShapeDtypeStruct((n, DIM), jnp.float32),
    )(h, aggnorm, w3, b3)


def _gather_rows(table, idx):
    return jnp.take(table, idx, axis=0)


def _segment_sum(vals, seg_ids, n_seg):
    return jax.ops.segment_sum(vals, seg_ids, num_segments=n_seg)


def _egconv(h, e, src, dst, W, b, n_seg, blk_node, blk_edge):
    w014 = jnp.stack([W[0], W[1], W[4]])
    b014 = jnp.stack([b[0], b[1], b[4]])
    p0, p1, p4 = _node_transform(h, w014, b014, blk_node)
    g0 = _gather_rows(p0, src)
    g1 = _gather_rows(p1, dst)
    g4 = _gather_rows(p4, src)
    m, e_new = _edge_update(g0, g1, g4, e, W[2], b[2], blk_edge)
    aggnorm = _segment_sum(m, dst, n_seg)
    h_new = _node_update(h, aggnorm, W[3], b[3], blk_node)
    return h_new, e_new


def kernel(x_atm, x_bnd, x_ang, edge_index_G, edge_index_A, x_atm_batch,
           embed_atm, Wa, ba, Wb, bb, W_head, b_head, W_out, b_out):
    N = x_atm.shape[0]
    E = x_bnd.shape[0]
    A = x_ang.shape[0]
    NI = Wa.shape[0]

    h_atm = jnp.take(embed_atm, x_atm, axis=0)

    narr = jnp.arange(1, DIM + 1, dtype=jnp.float32)
    xb = x_bnd
    h_bnd = (jnp.sqrt(2.0 / CUT) * jnp.sin(narr[None, :] * jnp.pi * xb[:, None] / CUT)
             / (xb[:, None] + 1e-9))

    mu = jnp.linspace(-1.0, 1.0, DIM, dtype=jnp.float32)
    step = 2.0 / (DIM - 1)
    ca = jnp.cos(x_ang)
    h_ang = jnp.exp(-((ca[:, None] - mu[None, :]) / step) ** 2)

    srcG, dstG = edge_index_G[0], edge_index_G[1]
    srcA, dstA = edge_index_A[0], edge_index_A[1]

    for i in range(NI):
        h_bnd, h_ang = _egconv(h_bnd, h_ang, srcA, dstA, Wb[i], bb[i], E,
                               blk_node=1000, blk_edge=1000)
        h_atm, h_bnd = _egconv(h_atm, h_bnd, srcG, dstG, Wa[i], ba[i], N,
                               blk_node=1000, blk_edge=1000)

    pooled = jax.ops.segment_sum(h_atm, x_atm_batch, num_segments=64)
    h = _silu(pooled @ W_head + b_head[None, :])
    return h @ W_out + b_out[None, :]


# speedup vs baseline: 1.3707x; 1.0983x over previous
import functools

import jax
import jax.numpy as jnp
from jax import lax
from jax.experimental import pallas as pl
from jax.experimental.pallas import tpu as pltpu
from jax.experimental.pallas import tpu_sc as plsc

DIM = 128
CUT = 3.0

_NC = 2
_NS = 16
_NW = _NC * _NS


def _sc_gather2(p04, p1, src, dst):
    B = src.shape[0]
    per = B // _NW
    W = 40
    nwin = per // W
    mesh = plsc.VectorSubcoreMesh(core_axis_name="c", subcore_axis_name="s",
                                  num_cores=_NC, num_subcores=_NS)

    def body(p04_ref, p1_ref, src_ref, dst_ref, o04_ref, o1_ref,
             idx_v, rows04, rows1, sem04, sem1):
        wid = lax.axis_index("s") * _NC + lax.axis_index("c")
        base = wid * per

        def phase(table, idxs_hbm, out_hbm, rows, sem):
            pltpu.sync_copy(idxs_hbm.at[pl.ds(base, per)], idx_v)

            def fire(g, slot):
                pltpu.make_async_copy(table.at[idx_v.at[pl.ds(g * W, W)]],
                                      rows.at[slot], sem).start()

            fire(0, 0)

            def step(g, carry):
                slot = lax.rem(g, 2)

                @pl.when(g + 1 < nwin)
                def _():
                    fire(g + 1, lax.rem(g + 1, 2))

                pltpu.make_async_copy(table.at[idx_v.at[pl.ds(g * W, W)]],
                                      rows.at[slot], sem).wait()
                pltpu.sync_copy(rows.at[slot],
                                out_hbm.at[pl.ds(base + g * W, W)])
                return carry

            lax.fori_loop(0, nwin, step, 0)

        phase(p04_ref, src_ref, o04_ref, rows04, sem04)
        phase(p1_ref, dst_ref, o1_ref, rows1, sem1)

    f = pl.kernel(
        body,
        out_type=[jax.ShapeDtypeStruct((B, 2 * DIM), jnp.float32),
                  jax.ShapeDtypeStruct((B, DIM), jnp.float32)],
        mesh=mesh,
        scratch_types=[
            pltpu.VMEM((per,), jnp.int32),
            pltpu.VMEM((2, W, 2 * DIM), jnp.float32),
            pltpu.VMEM((2, W, DIM), jnp.float32),
            pltpu.SemaphoreType.DMA,
            pltpu.SemaphoreType.DMA,
        ],
    )
    return f(p04, p1, src, dst)


def _silu(x):
    return x * jax.nn.sigmoid(x)


def _node_transform_body(h_ref, w_ref, b_ref, p04_ref, p1_ref):
    h = h_ref[...]
    w = w_ref[...]
    b = b_ref[...]
    p0 = jnp.dot(h, w[0], preferred_element_type=jnp.float32) + b[0][None, :]
    p4 = jnp.dot(h, w[2], preferred_element_type=jnp.float32) + b[2][None, :]
    p04_ref[...] = jnp.concatenate([p0, p4], axis=1)
    p1_ref[...] = jnp.dot(h, w[1], preferred_element_type=jnp.float32) + b[1][None, :]


def _node_transform(h, w3, b3, block):
    n = h.shape[0]
    grid = (n // block,)
    return pl.pallas_call(
        _node_transform_body,
        grid=grid,
        in_specs=[
            pl.BlockSpec((block, DIM), lambda i: (i, 0)),
            pl.BlockSpec((3, DIM, DIM), lambda i: (0, 0, 0)),
            pl.BlockSpec((3, DIM), lambda i: (0, 0)),
        ],
        out_specs=[
            pl.BlockSpec((block, 2 * DIM), lambda i: (i, 0)),
            pl.BlockSpec((block, DIM), lambda i: (i, 0)),
        ],
        out_shape=[
            jax.ShapeDtypeStruct((n, 2 * DIM), jnp.float32),
            jax.ShapeDtypeStruct((n, DIM), jnp.float32),
        ],
    )(h, w3, b3)


def _edge_update_body(g04_ref, g1_ref, e_ref, w2_ref, b2_ref,
                      m_ref, enew_ref):
    e = e_ref[...]
    g04 = g04_ref[...]
    z = (g04[:, :DIM] + g1_ref[...]
         + jnp.dot(e, w2_ref[...], preferred_element_type=jnp.float32)
         + b2_ref[...][None, :])
    sigma = jax.nn.sigmoid(z)
    m_ref[...] = jnp.concatenate([sigma * g04[:, DIM:], sigma], axis=1)
    enew_ref[...] = e + _silu(z)


def _edge_update(g04, g1, e, w2, b2, block):
    n = e.shape[0]
    grid = (n // block,)
    return pl.pallas_call(
        _edge_update_body,
        grid=grid,
        in_specs=[
            pl.BlockSpec((block, 2 * DIM), lambda i: (i, 0)),
            pl.BlockSpec((block, DIM), lambda i: (i, 0)),
            pl.BlockSpec((block, DIM), lambda i: (i, 0)),
            pl.BlockSpec((DIM, DIM), lambda i: (0, 0)),
            pl.BlockSpec((DIM,), lambda i: (0,)),
        ],
        out_specs=[
            pl.BlockSpec((block, 2 * DIM), lambda i: (i, 0)),
            pl.BlockSpec((block, DIM), lambda i: (i, 0)),
        ],
        out_shape=[
            jax.ShapeDtypeStruct((n, 2 * DIM), jnp.float32),
            jax.ShapeDtypeStruct((n, DIM), jnp.float32),
        ],
    )(g04, g1, e, w2, b2)


def _node_update_body(h_ref, an_ref, w3_ref, b3_ref, out_ref):
    h = h_ref[...]
    an = an_ref[...]
    agg = an[:, :DIM]
    norm = an[:, DIM:]
    s3 = jnp.dot(h, w3_ref[...], preferred_element_type=jnp.float32) + b3_ref[...][None, :]
    out_ref[...] = h + _silu(s3 + agg / (norm + 1e-6))


def _node_update(h, aggnorm, w3, b3, block):
    n = h.shape[0]
    grid = (n // block,)
    return pl.pallas_call(
        _node_update_body,
        grid=grid,
        in_specs=[
            pl.BlockSpec((block, DIM), lambda i: (i, 0)),
            pl.BlockSpec((block, 2 * DIM), lambda i: (i, 0)),
            pl.BlockSpec((DIM, DIM), lambda i: (0, 0)),
            pl.BlockSpec((DIM,), lambda i: (0,)),
        ],
        out_specs=pl.BlockSpec((block, DIM), lambda i: (i, 0)),
        out_shape=jax.ShapeDtypeStruct((n, DIM), jnp.float32),
    )(h, aggnorm, w3, b3)


def _gather_rows(table, idx):
    return jnp.take(table, idx, axis=0)


def _segment_sum(vals, seg_ids, n_seg):
    return jax.ops.segment_sum(vals, seg_ids, num_segments=n_seg)


def _egconv(h, e, src, dst, W, b, n_seg, blk_node, blk_edge):
    w014 = jnp.stack([W[0], W[1], W[4]])
    b014 = jnp.stack([b[0], b[1], b[4]])
    p04, p1 = _node_transform(h, w014, b014, blk_node)
    g04, g1 = _sc_gather2(p04, p1, src, dst)
    m, e_new = _edge_update(g04, g1, e, W[2], b[2], blk_edge)
    aggnorm = _segment_sum(m, dst, n_seg)
    h_new = _node_update(h, aggnorm, W[3], b[3], blk_node)
    return h_new, e_new


def kernel(x_atm, x_bnd, x_ang, edge_index_G, edge_index_A, x_atm_batch,
           embed_atm, Wa, ba, Wb, bb, W_head, b_head, W_out, b_out):
    N = x_atm.shape[0]
    E = x_bnd.shape[0]
    A = x_ang.shape[0]
    NI = Wa.shape[0]

    h_atm = jnp.take(embed_atm, x_atm, axis=0)

    narr = jnp.arange(1, DIM + 1, dtype=jnp.float32)
    xb = x_bnd
    h_bnd = (jnp.sqrt(2.0 / CUT) * jnp.sin(narr[None, :] * jnp.pi * xb[:, None] / CUT)
             / (xb[:, None] + 1e-9))

    mu = jnp.linspace(-1.0, 1.0, DIM, dtype=jnp.float32)
    step = 2.0 / (DIM - 1)
    ca = jnp.cos(x_ang)
    h_ang = jnp.exp(-((ca[:, None] - mu[None, :]) / step) ** 2)

    srcG, dstG = edge_index_G[0], edge_index_G[1]
    srcA, dstA = edge_index_A[0], edge_index_A[1]

    for i in range(NI):
        h_bnd, h_ang = _egconv(h_bnd, h_ang, srcA, dstA, Wb[i], bb[i], E,
                               blk_node=1000, blk_edge=1000)
        h_atm, h_bnd = _egconv(h_atm, h_bnd, srcG, dstG, Wa[i], ba[i], N,
                               blk_node=1000, blk_edge=1000)

    pooled = jax.ops.segment_sum(h_atm, x_atm_batch, num_segments=64)
    h = _silu(pooled @ W_head + b_head[None, :])
    return h @ W_out + b_out[None, :]


# speedup vs baseline: 1.3806x; 1.0072x over previous
import functools

import jax
import jax.numpy as jnp
from jax import lax
from jax.experimental import pallas as pl
from jax.experimental.pallas import tpu as pltpu
from jax.experimental.pallas import tpu_sc as plsc

DIM = 128
CUT = 3.0

_NC = 2
_NS = 16
_NW = _NC * _NS


def _sc_gather2(p04, p1, src, dst):
    B = src.shape[0]
    per = B // _NW
    W = 40
    nwin = per // W
    mesh = plsc.VectorSubcoreMesh(core_axis_name="c", subcore_axis_name="s",
                                  num_cores=_NC, num_subcores=_NS)

    def body(p04_ref, p1_ref, src_ref, dst_ref, o04_ref, o1_ref,
             idx_v, rows04, rows1, sem04, sem1):
        wid = lax.axis_index("s") * _NC + lax.axis_index("c")
        base = wid * per

        def phase(table, idxs_hbm, out_hbm, rows, sem):
            pltpu.sync_copy(idxs_hbm.at[pl.ds(base, per)], idx_v)

            def fire(g, slot):
                pltpu.make_async_copy(table.at[idx_v.at[pl.ds(g * W, W)]],
                                      rows.at[slot], sem).start()

            fire(0, 0)

            def step(g, carry):
                slot = lax.rem(g, 2)

                @pl.when(g + 1 < nwin)
                def _():
                    fire(g + 1, lax.rem(g + 1, 2))

                pltpu.make_async_copy(table.at[idx_v.at[pl.ds(g * W, W)]],
                                      rows.at[slot], sem).wait()
                pltpu.sync_copy(rows.at[slot],
                                out_hbm.at[pl.ds(base + g * W, W)])
                return carry

            lax.fori_loop(0, nwin, step, 0)

        phase(p04_ref, src_ref, o04_ref, rows04, sem04)
        phase(p1_ref, dst_ref, o1_ref, rows1, sem1)

    f = pl.kernel(
        body,
        out_type=[jax.ShapeDtypeStruct((B, 2 * DIM), jnp.float32),
                  jax.ShapeDtypeStruct((B, DIM), jnp.float32)],
        mesh=mesh,
        scratch_types=[
            pltpu.VMEM((per,), jnp.int32),
            pltpu.VMEM((2, W, 2 * DIM), jnp.float32),
            pltpu.VMEM((2, W, DIM), jnp.float32),
            pltpu.SemaphoreType.DMA,
            pltpu.SemaphoreType.DMA,
        ],
    )
    return f(p04, p1, src, dst)


def _silu(x):
    return x * jax.nn.sigmoid(x)


def _node_transform_body(h_ref, w_ref, b_ref, p04_ref, p1_ref):
    h = h_ref[...]
    w = w_ref[...]
    b = b_ref[...]
    p0 = jnp.dot(h, w[0], preferred_element_type=jnp.float32) + b[0][None, :]
    p4 = jnp.dot(h, w[2], preferred_element_type=jnp.float32) + b[2][None, :]
    p04_ref[...] = jnp.concatenate([p0, p4], axis=1)
    p1_ref[...] = jnp.dot(h, w[1], preferred_element_type=jnp.float32) + b[1][None, :]


def _node_transform(h, w3, b3, block):
    n = h.shape[0]
    grid = (n // block,)
    return pl.pallas_call(
        _node_transform_body,
        grid=grid,
        in_specs=[
            pl.BlockSpec((block, DIM), lambda i: (i, 0)),
            pl.BlockSpec((3, DIM, DIM), lambda i: (0, 0, 0)),
            pl.BlockSpec((3, DIM), lambda i: (0, 0)),
        ],
        out_specs=[
            pl.BlockSpec((block, 2 * DIM), lambda i: (i, 0)),
            pl.BlockSpec((block, DIM), lambda i: (i, 0)),
        ],
        out_shape=[
            jax.ShapeDtypeStruct((n, 2 * DIM), jnp.float32),
            jax.ShapeDtypeStruct((n, DIM), jnp.float32),
        ],
    )(h, w3, b3)


def _edge_update_body(g04_ref, g1_ref, e_ref, w2_ref, b2_ref,
                      m_ref, enew_ref):
    e = e_ref[...]
    g04 = g04_ref[...]
    z = (g04[:, :DIM] + g1_ref[...]
         + jnp.dot(e, w2_ref[...], preferred_element_type=jnp.float32)
         + b2_ref[...][None, :])
    sigma = jax.nn.sigmoid(z)
    m_ref[...] = jnp.concatenate([sigma * g04[:, DIM:], sigma], axis=1)
    enew_ref[...] = e + _silu(z)


def _edge_update(g04, g1, e, w2, b2, block):
    n = e.shape[0]
    grid = (n // block,)
    return pl.pallas_call(
        _edge_update_body,
        grid=grid,
        in_specs=[
            pl.BlockSpec((block, 2 * DIM), lambda i: (i, 0)),
            pl.BlockSpec((block, DIM), lambda i: (i, 0)),
            pl.BlockSpec((block, DIM), lambda i: (i, 0)),
            pl.BlockSpec((DIM, DIM), lambda i: (0, 0)),
            pl.BlockSpec((DIM,), lambda i: (0,)),
        ],
        out_specs=[
            pl.BlockSpec((block, 2 * DIM), lambda i: (i, 0)),
            pl.BlockSpec((block, DIM), lambda i: (i, 0)),
        ],
        out_shape=[
            jax.ShapeDtypeStruct((n, 2 * DIM), jnp.float32),
            jax.ShapeDtypeStruct((n, DIM), jnp.float32),
        ],
    )(g04, g1, e, w2, b2)


def _node_update_body(h_ref, an_ref, w3_ref, b3_ref, out_ref):
    h = h_ref[...]
    an = an_ref[...]
    agg = an[:, :DIM]
    norm = an[:, DIM:]
    s3 = jnp.dot(h, w3_ref[...], preferred_element_type=jnp.float32) + b3_ref[...][None, :]
    out_ref[...] = h + _silu(s3 + agg / (norm + 1e-6))


def _node_update(h, aggnorm, w3, b3, block):
    n = h.shape[0]
    grid = (n // block,)
    return pl.pallas_call(
        _node_update_body,
        grid=grid,
        in_specs=[
            pl.BlockSpec((block, DIM), lambda i: (i, 0)),
            pl.BlockSpec((block, 2 * DIM), lambda i: (i, 0)),
            pl.BlockSpec((DIM, DIM), lambda i: (0, 0)),
            pl.BlockSpec((DIM,), lambda i: (0,)),
        ],
        out_specs=pl.BlockSpec((block, DIM), lambda i: (i, 0)),
        out_shape=jax.ShapeDtypeStruct((n, DIM), jnp.float32),
    )(h, aggnorm, w3, b3)


def _embed_body(nspec, ids_ref, tbl_ref, out_ref):
    ids = ids_ref[0]
    iota = jax.lax.broadcasted_iota(jnp.int32, (1000, nspec), 1)
    onehot = (ids[0][:, None] == iota).astype(jnp.float32)
    out_ref[...] = jnp.dot(onehot, tbl_ref[...],
                           preferred_element_type=jnp.float32)


def _tc_embed(x_atm, embed_atm):
    n = x_atm.shape[0]
    nspec = embed_atm.shape[0]
    ids3 = x_atm.astype(jnp.int32).reshape(n // 1000, 1, 1000)
    return pl.pallas_call(
        functools.partial(_embed_body, nspec),
        grid=(n // 1000,),
        in_specs=[
            pl.BlockSpec((1, 1, 1000), lambda i: (i, 0, 0)),
            pl.BlockSpec((nspec, DIM), lambda i: (0, 0)),
        ],
        out_specs=pl.BlockSpec((1000, DIM), lambda i: (i, 0)),
        out_shape=jax.ShapeDtypeStruct((n, DIM), jnp.float32),
    )(ids3, embed_atm)


def _pool_head_body(ng, h_ref, b_ref, wh_ref, bh_ref, wo_ref, bo_ref,
                    out_ref, acc_ref):
    i = pl.program_id(0)

    @pl.when(i == 0)
    def _():
        acc_ref[...] = jnp.zeros_like(acc_ref)

    ids = b_ref[0]
    iota = jax.lax.broadcasted_iota(jnp.int32, (ng, 1000), 0)
    onehot = (ids[0][None, :] == iota).astype(jnp.float32)
    acc_ref[...] += jnp.dot(onehot, h_ref[...],
                            preferred_element_type=jnp.float32)

    @pl.when(i == pl.num_programs(0) - 1)
    def _():
        hh = _silu(jnp.dot(acc_ref[...], wh_ref[...],
                           preferred_element_type=jnp.float32)
                   + bh_ref[...][None, :])
        out_ref[...] = jnp.dot(hh, wo_ref[...],
                               preferred_element_type=jnp.float32) + bo_ref[...][None, :]


def _tc_pool_head(h_atm, batch, w_head, b_head, w_out_pad, b_out_pad, ng):
    n = h_atm.shape[0]
    b3 = batch.astype(jnp.int32).reshape(n // 1000, 1, 1000)
    return pl.pallas_call(
        functools.partial(_pool_head_body, ng),
        grid=(n // 1000,),
        in_specs=[
            pl.BlockSpec((1000, DIM), lambda i: (i, 0)),
            pl.BlockSpec((1, 1, 1000), lambda i: (i, 0, 0)),
            pl.BlockSpec((DIM, DIM), lambda i: (0, 0)),
            pl.BlockSpec((DIM,), lambda i: (0,)),
            pl.BlockSpec((DIM, DIM), lambda i: (0, 0)),
            pl.BlockSpec((DIM,), lambda i: (0,)),
        ],
        out_specs=pl.BlockSpec((ng, DIM), lambda i: (0, 0)),
        out_shape=jax.ShapeDtypeStruct((ng, DIM), jnp.float32),
        scratch_shapes=[pltpu.VMEM((ng, DIM), jnp.float32)],
    )(h_atm, b3, w_head, b_head, w_out_pad, b_out_pad)


def _gather_rows(table, idx):
    return jnp.take(table, idx, axis=0)


def _segment_sum(vals, seg_ids, n_seg):
    return jax.ops.segment_sum(vals, seg_ids, num_segments=n_seg)


def _egconv(h, e, src, dst, W, b, n_seg, blk_node, blk_edge):
    w014 = jnp.stack([W[0], W[1], W[4]])
    b014 = jnp.stack([b[0], b[1], b[4]])
    p04, p1 = _node_transform(h, w014, b014, blk_node)
    g04, g1 = _sc_gather2(p04, p1, src, dst)
    m, e_new = _edge_update(g04, g1, e, W[2], b[2], blk_edge)
    aggnorm = _segment_sum(m, dst, n_seg)
    h_new = _node_update(h, aggnorm, W[3], b[3], blk_node)
    return h_new, e_new


def kernel(x_atm, x_bnd, x_ang, edge_index_G, edge_index_A, x_atm_batch,
           embed_atm, Wa, ba, Wb, bb, W_head, b_head, W_out, b_out):
    N = x_atm.shape[0]
    E = x_bnd.shape[0]
    A = x_ang.shape[0]
    NI = Wa.shape[0]

    h_atm = _tc_embed(x_atm, embed_atm)

    narr = jnp.arange(1, DIM + 1, dtype=jnp.float32)
    xb = x_bnd
    h_bnd = (jnp.sqrt(2.0 / CUT) * jnp.sin(narr[None, :] * jnp.pi * xb[:, None] / CUT)
             / (xb[:, None] + 1e-9))

    mu = jnp.linspace(-1.0, 1.0, DIM, dtype=jnp.float32)
    step = 2.0 / (DIM - 1)
    ca = jnp.cos(x_ang)
    h_ang = jnp.exp(-((ca[:, None] - mu[None, :]) / step) ** 2)

    srcG, dstG = edge_index_G[0], edge_index_G[1]
    srcA, dstA = edge_index_A[0], edge_index_A[1]

    for i in range(NI):
        h_bnd, h_ang = _egconv(h_bnd, h_ang, srcA, dstA, Wb[i], bb[i], E,
                               blk_node=1000, blk_edge=1000)
        h_atm, h_bnd = _egconv(h_atm, h_bnd, srcG, dstG, Wa[i], ba[i], N,
                               blk_node=1000, blk_edge=1000)

    w_out_pad = jnp.zeros((DIM, DIM), jnp.float32).at[:, :3].set(W_out)
    b_out_pad = jnp.zeros((DIM,), jnp.float32).at[:3].set(b_out)
    out = _tc_pool_head(h_atm, x_atm_batch, W_head, b_head,
                        w_out_pad, b_out_pad, 64)
    return out[:, :3]


# speedup vs baseline: 1.3883x; 1.0056x over previous
import functools

import jax
import jax.numpy as jnp
from jax import lax
from jax.experimental import pallas as pl
from jax.experimental.pallas import tpu as pltpu
from jax.experimental.pallas import tpu_sc as plsc

DIM = 128
CUT = 3.0

_NC = 2
_NS = 16
_NW = _NC * _NS


def _sc_gather2(p04, p1, src, dst):
    B = src.shape[0]
    per = B // _NW
    W04 = 200
    W1 = 40
    mesh = plsc.VectorSubcoreMesh(core_axis_name="c", subcore_axis_name="s",
                                  num_cores=_NC, num_subcores=_NS)

    def body(p04_ref, p1_ref, src_ref, dst_ref, o04_ref, o1_ref,
             idx_v, rows04, rows1, gsem, osem):
        wid = lax.axis_index("s") * _NC + lax.axis_index("c")
        base = wid * per

        def phase(table, idxs_hbm, out_hbm, rows, W, width):
            nwin = per // W
            pltpu.sync_copy(idxs_hbm.at[pl.ds(base, per)], idx_v)

            def g_desc(g, slot):
                return pltpu.make_async_copy(
                    table.at[idx_v.at[pl.ds(g * W, W)]], rows.at[slot], gsem)

            def o_desc(g, slot):
                return pltpu.make_async_copy(
                    rows.at[slot], out_hbm.at[pl.ds(base + g * W, W)], osem)

            g_desc(0, 0).start()

            def step(g, carry):
                slot = lax.rem(g, 2)
                nslot = lax.rem(g + 1, 2)

                @pl.when(g + 1 < nwin)
                def _():
                    @pl.when(g >= 1)
                    def _():
                        o_desc(g - 1, nslot).wait()

                    g_desc(g + 1, nslot).start()

                g_desc(g, slot).wait()
                o_desc(g, slot).start()
                return carry

            lax.fori_loop(0, nwin, step, 0)
            o_desc(nwin - 2, lax.rem(nwin - 2, 2)).wait()
            o_desc(nwin - 1, lax.rem(nwin - 1, 2)).wait()

        phase(p04_ref, src_ref, o04_ref, rows04, W04, 2 * DIM)
        phase(p1_ref, dst_ref, o1_ref, rows1, W1, DIM)

    f = pl.kernel(
        body,
        out_type=[jax.ShapeDtypeStruct((B, 2 * DIM), jnp.float32),
                  jax.ShapeDtypeStruct((B, DIM), jnp.float32)],
        mesh=mesh,
        scratch_types=[
            pltpu.VMEM((per,), jnp.int32),
            pltpu.VMEM((2, W04, 2 * DIM), jnp.float32),
            pltpu.VMEM((2, W1, DIM), jnp.float32),
            pltpu.SemaphoreType.DMA,
            pltpu.SemaphoreType.DMA,
        ],
    )
    return f(p04, p1, src, dst)


def _silu(x):
    return x * jax.nn.sigmoid(x)


def _node_transform_body(h_ref, w_ref, b_ref, p04_ref, p1_ref):
    h = h_ref[...]
    w = w_ref[...]
    b = b_ref[...]
    p0 = jnp.dot(h, w[0], preferred_element_type=jnp.float32) + b[0][None, :]
    p4 = jnp.dot(h, w[2], preferred_element_type=jnp.float32) + b[2][None, :]
    p04_ref[...] = jnp.concatenate([p0, p4], axis=1)
    p1_ref[...] = jnp.dot(h, w[1], preferred_element_type=jnp.float32) + b[1][None, :]


def _node_transform(h, w3, b3, block):
    n = h.shape[0]
    grid = (n // block,)
    return pl.pallas_call(
        _node_transform_body,
        grid=grid,
        in_specs=[
            pl.BlockSpec((block, DIM), lambda i: (i, 0)),
            pl.BlockSpec((3, DIM, DIM), lambda i: (0, 0, 0)),
            pl.BlockSpec((3, DIM), lambda i: (0, 0)),
        ],
        out_specs=[
            pl.BlockSpec((block, 2 * DIM), lambda i: (i, 0)),
            pl.BlockSpec((block, DIM), lambda i: (i, 0)),
        ],
        out_shape=[
            jax.ShapeDtypeStruct((n, 2 * DIM), jnp.float32),
            jax.ShapeDtypeStruct((n, DIM), jnp.float32),
        ],
    )(h, w3, b3)


def _edge_update_body(g04_ref, g1_ref, e_ref, w2_ref, b2_ref,
                      m_ref, enew_ref):
    e = e_ref[...]
    g04 = g04_ref[...]
    z = (g04[:, :DIM] + g1_ref[...]
         + jnp.dot(e, w2_ref[...], preferred_element_type=jnp.float32)
         + b2_ref[...][None, :])
    sigma = jax.nn.sigmoid(z)
    m_ref[...] = jnp.concatenate([sigma * g04[:, DIM:], sigma], axis=1)
    enew_ref[...] = e + _silu(z)


def _edge_update(g04, g1, e, w2, b2, block):
    n = e.shape[0]
    grid = (n // block,)
    return pl.pallas_call(
        _edge_update_body,
        grid=grid,
        in_specs=[
            pl.BlockSpec((block, 2 * DIM), lambda i: (i, 0)),
            pl.BlockSpec((block, DIM), lambda i: (i, 0)),
            pl.BlockSpec((block, DIM), lambda i: (i, 0)),
            pl.BlockSpec((DIM, DIM), lambda i: (0, 0)),
            pl.BlockSpec((DIM,), lambda i: (0,)),
        ],
        out_specs=[
            pl.BlockSpec((block, 2 * DIM), lambda i: (i, 0)),
            pl.BlockSpec((block, DIM), lambda i: (i, 0)),
        ],
        out_shape=[
            jax.ShapeDtypeStruct((n, 2 * DIM), jnp.float32),
            jax.ShapeDtypeStruct((n, DIM), jnp.float32),
        ],
    )(g04, g1, e, w2, b2)


def _node_update_body(h_ref, an_ref, w3_ref, b3_ref, out_ref):
    h = h_ref[...]
    an = an_ref[...]
    agg = an[:, :DIM]
    norm = an[:, DIM:]
    s3 = jnp.dot(h, w3_ref[...], preferred_element_type=jnp.float32) + b3_ref[...][None, :]
    out_ref[...] = h + _silu(s3 + agg / (norm + 1e-6))


def _node_update(h, aggnorm, w3, b3, block):
    n = h.shape[0]
    grid = (n // block,)
    return pl.pallas_call(
        _node_update_body,
        grid=grid,
        in_specs=[
            pl.BlockSpec((block, DIM), lambda i: (i, 0)),
            pl.BlockSpec((block, 2 * DIM), lambda i: (i, 0)),
            pl.BlockSpec((DIM, DIM), lambda i: (0, 0)),
            pl.BlockSpec((DIM,), lambda i: (0,)),
        ],
        out_specs=pl.BlockSpec((block, DIM), lambda i: (i, 0)),
        out_shape=jax.ShapeDtypeStruct((n, DIM), jnp.float32),
    )(h, aggnorm, w3, b3)


def _embed_body(nspec, ids_ref, tbl_ref, out_ref):
    ids = ids_ref[0]
    iota = jax.lax.broadcasted_iota(jnp.int32, (1000, nspec), 1)
    onehot = (ids[0][:, None] == iota).astype(jnp.float32)
    out_ref[...] = jnp.dot(onehot, tbl_ref[...],
                           preferred_element_type=jnp.float32,
                           precision=jax.lax.Precision.HIGHEST)


def _tc_embed(x_atm, embed_atm):
    n = x_atm.shape[0]
    nspec = embed_atm.shape[0]
    ids3 = x_atm.astype(jnp.int32).reshape(n // 1000, 1, 1000)
    return pl.pallas_call(
        functools.partial(_embed_body, nspec),
        grid=(n // 1000,),
        in_specs=[
            pl.BlockSpec((1, 1, 1000), lambda i: (i, 0, 0)),
            pl.BlockSpec((nspec, DIM), lambda i: (0, 0)),
        ],
        out_specs=pl.BlockSpec((1000, DIM), lambda i: (i, 0)),
        out_shape=jax.ShapeDtypeStruct((n, DIM), jnp.float32),
    )(ids3, embed_atm)


def _pool_head_body(ng, h_ref, b_ref, wh_ref, bh_ref, wo_ref, bo_ref,
                    out_ref, acc_ref):
    i = pl.program_id(0)

    @pl.when(i == 0)
    def _():
        acc_ref[...] = jnp.zeros_like(acc_ref)

    ids = b_ref[0]
    iota = jax.lax.broadcasted_iota(jnp.int32, (ng, 1000), 0)
    onehot = (ids[0][None, :] == iota).astype(jnp.float32)
    acc_ref[...] += jnp.dot(onehot, h_ref[...],
                            preferred_element_type=jnp.float32,
                            precision=jax.lax.Precision.HIGHEST)

    @pl.when(i == pl.num_programs(0) - 1)
    def _():
        hh = _silu(jnp.dot(acc_ref[...], wh_ref[...],
                           preferred_element_type=jnp.float32,
                           precision=jax.lax.Precision.HIGHEST)
                   + bh_ref[...][None, :])
        out_ref[...] = jnp.dot(hh, wo_ref[...],
                               preferred_element_type=jnp.float32,
                               precision=jax.lax.Precision.HIGHEST) + bo_ref[...][None, :]


def _tc_pool_head(h_atm, batch, w_head, b_head, w_out_pad, b_out_pad, ng):
    n = h_atm.shape[0]
    b3 = batch.astype(jnp.int32).reshape(n // 1000, 1, 1000)
    return pl.pallas_call(
        functools.partial(_pool_head_body, ng),
        grid=(n // 1000,),
        in_specs=[
            pl.BlockSpec((1000, DIM), lambda i: (i, 0)),
            pl.BlockSpec((1, 1, 1000), lambda i: (i, 0, 0)),
            pl.BlockSpec((DIM, DIM), lambda i: (0, 0)),
            pl.BlockSpec((DIM,), lambda i: (0,)),
            pl.BlockSpec((DIM, DIM), lambda i: (0, 0)),
            pl.BlockSpec((DIM,), lambda i: (0,)),
        ],
        out_specs=pl.BlockSpec((ng, DIM), lambda i: (0, 0)),
        out_shape=jax.ShapeDtypeStruct((ng, DIM), jnp.float32),
        scratch_shapes=[pltpu.VMEM((ng, DIM), jnp.float32)],
    )(h_atm, b3, w_head, b_head, w_out_pad, b_out_pad)


def _gather_rows(table, idx):
    return jnp.take(table, idx, axis=0)


def _segment_sum(vals, seg_ids, n_seg):
    return jax.ops.segment_sum(vals, seg_ids, num_segments=n_seg)


def _egconv(h, e, src, dst, W, b, n_seg, blk_node, blk_edge):
    w014 = jnp.stack([W[0], W[1], W[4]])
    b014 = jnp.stack([b[0], b[1], b[4]])
    p04, p1 = _node_transform(h, w014, b014, blk_node)
    g04, g1 = _sc_gather2(p04, p1, src, dst)
    m, e_new = _edge_update(g04, g1, e, W[2], b[2], blk_edge)
    aggnorm = _segment_sum(m, dst, n_seg)
    h_new = _node_update(h, aggnorm, W[3], b[3], blk_node)
    return h_new, e_new


def kernel(x_atm, x_bnd, x_ang, edge_index_G, edge_index_A, x_atm_batch,
           embed_atm, Wa, ba, Wb, bb, W_head, b_head, W_out, b_out):
    N = x_atm.shape[0]
    E = x_bnd.shape[0]
    A = x_ang.shape[0]
    NI = Wa.shape[0]

    h_atm = _tc_embed(x_atm, embed_atm)

    narr = jnp.arange(1, DIM + 1, dtype=jnp.float32)
    xb = x_bnd
    h_bnd = (jnp.sqrt(2.0 / CUT) * jnp.sin(narr[None, :] * jnp.pi * xb[:, None] / CUT)
             / (xb[:, None] + 1e-9))

    mu = jnp.linspace(-1.0, 1.0, DIM, dtype=jnp.float32)
    step = 2.0 / (DIM - 1)
    ca = jnp.cos(x_ang)
    h_ang = jnp.exp(-((ca[:, None] - mu[None, :]) / step) ** 2)

    srcG, dstG = edge_index_G[0], edge_index_G[1]
    srcA, dstA = edge_index_A[0], edge_index_A[1]

    for i in range(NI):
        h_bnd, h_ang = _egconv(h_bnd, h_ang, srcA, dstA, Wb[i], bb[i], E,
                               blk_node=1000, blk_edge=1000)
        h_atm, h_bnd = _egconv(h_atm, h_bnd, srcG, dstG, Wa[i], ba[i], N,
                               blk_node=1000, blk_edge=1000)

    w_out_pad = jnp.zeros((DIM, DIM), jnp.float32).at[:, :3].set(W_out)
    b_out_pad = jnp.zeros((DIM,), jnp.float32).at[:3].set(b_out)
    out = _tc_pool_head(h_atm, x_atm_batch, W_head, b_head,
                        w_out_pad, b_out_pad, 64)
    return out[:, :3]
